# baseline (speedup 1.0000x reference)
import functools

import jax
import jax.numpy as jnp
from jax import lax
from jax.experimental import pallas as pl
from jax.experimental.pallas import tpu as pltpu
from jax.experimental.pallas import tpu_sc as plsc

_N = 10000
_E = 320000
_H = 128
_OUT = 40
_POOL = 4

_NC = 2
_NS = 16
_NW = _NC * _NS
_CH = 128
_CPT = 80
_CPH = 40
_NCH = _NW * _CPT
_EP = _NCH * _CH
_NP = 10240
_ROWS_PER_TILE = _NP // _NS
_NPAD = _NP - _N

_BR = 1024
_G = _NP // _BR
_BC = _BR // _POOL


def _sc_mesh():
    return plsc.VectorSubcoreMesh(core_axis_name="c", subcore_axis_name="s")


def _deg_partials(dst2d, ones_rows, zero_rows):
    @functools.partial(
        pl.kernel,
        mesh=_sc_mesh(),
        out_type=jax.ShapeDtypeStruct((_NC, _NP, _H), jnp.float32),
        scratch_types=[
            pltpu.VMEM_SHARED((_NP, _H), jnp.float32),
            pltpu.VMEM((_CPT, _CH), jnp.int32),
            pltpu.VMEM((_CH, _H), jnp.float32),
        ],
    )
    def k(dst_hbm, ones_hbm, z_hbm, out_hbm, acc, dst_v, ones_v):
        c = lax.axis_index("c")
        s = lax.axis_index("s")
        w = s * _NC + c
        row0 = s * _ROWS_PER_TILE
        ch0 = w * _CPT
        pltpu.sync_copy(z_hbm, acc.at[pl.ds(row0, _ROWS_PER_TILE)])
        pltpu.sync_copy(dst_hbm.at[pl.ds(ch0, _CPT)], dst_v)
        pltpu.sync_copy(ones_hbm, ones_v)
        plsc.subcore_barrier()

        def body(j, carry):
            pltpu.sync_copy(ones_v, acc.at[dst_v.at[j]], add=True)
            return carry

        lax.fori_loop(0, _CPT, body, 0)
        plsc.subcore_barrier()
        pltpu.sync_copy(acc.at[pl.ds(row0, _ROWS_PER_TILE)],
                        out_hbm.at[c, pl.ds(row0, _ROWS_PER_TILE)])

    return k(dst2d, ones_rows, zero_rows)


def _agg_partials(hn, src2d, dst2d, zero_rows):
    @functools.partial(
        pl.kernel,
        mesh=_sc_mesh(),
        out_type=jax.ShapeDtypeStruct((_NC, _NP, _H), jnp.float32),
        scratch_types=[
            pltpu.VMEM_SHARED((_NP, _H), jnp.float32),
            pltpu.VMEM((_CPH, _CH), jnp.int32),
            pltpu.VMEM((_CPH, _CH), jnp.int32),
            pltpu.VMEM((_CH, _H), jnp.float32),
            pltpu.VMEM((_CH, _H), jnp.float32),
            pltpu.SemaphoreType.DMA,
            pltpu.SemaphoreType.DMA,
        ],
    )
    def k(hn_hbm, src_hbm, dst_hbm, z_hbm, out_hbm, acc, src_v, dst_v,
          rows0, rows1, sem0, sem1):
        c = lax.axis_index("c")
        s = lax.axis_index("s")
        w = s * _NC + c
        row0 = s * _ROWS_PER_TILE

        rows = (rows0, rows1)
        sems = (sem0, sem1)

        def gath(j, b):
            return pltpu.make_async_copy(hn_hbm.at[src_v.at[j]], rows[b],
                                         sems[b])

        for ph in range(2):
            ch0 = w * _CPT + ph * _CPH
            pltpu.sync_copy(src_hbm.at[pl.ds(ch0, _CPH)], src_v)
            pltpu.sync_copy(dst_hbm.at[pl.ds(ch0, _CPH)], dst_v)

            gath(0, 0).start()
            gath(1, 1).start()

            if ph == 0:
                pltpu.sync_copy(z_hbm, acc.at[pl.ds(row0, _ROWS_PER_TILE)])
                plsc.subcore_barrier()

            def body(i, carry):
                for b in range(2):
                    j = 2 * i + b
                    gath(j, b).wait()
                    pltpu.sync_copy(rows[b], acc.at[dst_v.at[j]], add=True)

                    @pl.when(j + 2 < _CPH)
                    def _():
                        gath(j + 2, b).start()

                return carry

            lax.fori_loop(0, _CPH // 2, body, 0)
        plsc.subcore_barrier()
        pltpu.sync_copy(acc.at[pl.ds(row0, _ROWS_PER_TILE)],
                        out_hbm.at[c, pl.ds(row0, _ROWS_PER_TILE)])

    return k(hn, src2d, dst2d, zero_rows)


def _row_spec(d):
    return pl.BlockSpec((_BR, d), lambda i: (i, 0))


def _plane_spec(d, c):
    return pl.BlockSpec((1, _BR, d), lambda i, _c=c: (_c, i, 0))


def _full_spec(shape):
    nd = len(shape)
    return pl.BlockSpec(shape, lambda i: (0,) * nd)


def _tc_call(body, out_shapes, row_args, full_args):
    in_specs = []
    arrs = []
    for a in row_args:
        if isinstance(a, tuple):
            arr, plane = a
            in_specs.append(_plane_spec(arr.shape[2], plane))
            arrs.append(arr)
        else:
            in_specs.append(_row_spec(a.shape[1]))
            arrs.append(a)
    row_args = arrs
    in_specs += [_full_spec(a.shape) for a in full_args]
    out_specs = [_row_spec(s[1]) for s in out_shapes]
    out_shape = [jax.ShapeDtypeStruct(s, jnp.float32) for s in out_shapes]
    single = len(out_shapes) == 1
    res = pl.pallas_call(
        body,
        grid=(_G,),
        in_specs=in_specs,
        out_specs=out_specs[0] if single else out_specs,
        out_shape=out_shape[0] if single else out_shape,
        compiler_params=pltpu.CompilerParams(
            dimension_semantics=("arbitrary",)),
    )(*row_args, *full_args)
    return res


def _nrm(d0, d1):
    return lax.rsqrt(d0[0, :, 0:1] + d1[0, :, 0:1] + 1.0)


def _psum(p0, p1):
    return p0[0] + p1[0]


def _mm(a, b):
    return jnp.dot(a, b, preferred_element_type=jnp.float32)


def _t1a_body(x, W1, b1, W2, b2, h0_o):
    t = jnp.maximum(_mm(x[...], W1[...]) + b1[...], 0.0)
    h0_o[...] = _mm(t, W2[...]) + b2[...]


def _t1b_body(h0, d0, d1, hn_o):
    hn_o[...] = h0[...] * _nrm(d0[...], d1[...])


def _t2_body(p0, p1, d0, d1, W, b, hn_o):
    nrm = _nrm(d0[...], d1[...])
    agg = _psum(p0[...], p1[...]) * nrm
    h = jnp.maximum(_mm(agg, W[...]) + b[...], 0.0)
    hn_o[...] = h * nrm


def _t3_body(p0, p1, d0, d1, h0, Wg2, bg2, ws, Wdp, bdp, Wdq, bdq, P, PT,
             hn_o):
    nrm = _nrm(d0[...], d1[...])
    agg = _psum(p0[...], p1[...]) * nrm
    hg = jnp.maximum(_mm(agg, Wg2[...]) + bg2[...], 0.0)
    sc = _mm(hg, ws[...])
    ns = 1.0 / (1.0 + jnp.exp(-sc))
    hd = jnp.maximum(_mm(hg, Wdp[...]) + bdp[...], 0.0)
    hp_pre = _mm(P[...], hd)
    ps = _mm(P[...], ns) * 0.25
    hp = jnp.maximum(_mm(hp_pre, Wdq[...]) + bdq[...], 0.0) * ps
    hu = _mm(PT[...], hp) + h0[...]
    hn_o[...] = hu * nrm


def _t5_body(p0, p1, d0, d1, W, b, Wp1, bp1, Wp2, bp2, out_o):
    nrm = _nrm(d0[...], d1[...])
    agg = _psum(p0[...], p1[...]) * nrm
    h = jnp.maximum(_mm(agg, W[...]) + b[...], 0.0)
    t = jnp.maximum(_mm(h, Wp1[...]) + bp1[...], 0.0)
    out_o[...] = _mm(t, Wp2[...]) + bp2[...]


def kernel(x, edge_index, W1, b1, W2, b2, Wg1, bg1, Wg2, bg2, w_score,
           Wdp, bdp, Wdq, bdq, Wp1, bp1, Wp2, bp2):
    src = edge_index[0].astype(jnp.int32)
    dst = edge_index[1].astype(jnp.int32)

    pad = _N + (jnp.arange(_EP - _E, dtype=jnp.int32) % _NPAD)
    src2d = jnp.concatenate([src, pad]).reshape(_NCH, _CH)
    dst2d = jnp.concatenate([dst, pad]).reshape(_NCH, _CH)

    ones_rows = jnp.ones((_CH, _H), jnp.float32)
    zero128 = jnp.zeros((_ROWS_PER_TILE, _H), jnp.float32)

    rows = jnp.arange(_BC)[:, None]
    cols = jnp.arange(_BR)[None, :]
    P = (cols // _POOL == rows).astype(jnp.float32)
    PT = P.T

    b1r = b1.reshape(1, _H)
    b2r = b2.reshape(1, _H)
    bg1r = bg1.reshape(1, _H)
    bg2r = bg2.reshape(1, _H)
    bdpr = bdp.reshape(1, _H)
    bdqr = bdq.reshape(1, _H)
    bp1r = bp1.reshape(1, _H)
    bp2r = bp2.reshape(1, _OUT)
    wsr = w_score.reshape(_H, 1)

    h0 = _tc_call(_t1a_body, [(_NP, _H)], [x], [W1, b1r, W2, b2r])

    degp = _deg_partials(dst2d, ones_rows, zero128)
    degs = degp[:, :, :8]
    dd = [(degs, 0), (degs, 1)]

    hn0 = _tc_call(_t1b_body, [(_NP, _H)], [h0] + dd, [])

    p = _agg_partials(hn0, src2d, dst2d, zero128)
    hn1 = _tc_call(_t2_body, [(_NP, _H)],
                   [(p, 0), (p, 1)] + dd, [Wg1, bg1r])

    p = _agg_partials(hn1, src2d, dst2d, zero128)
    hnu = _tc_call(_t3_body, [(_NP, _H)],
                   [(p, 0), (p, 1)] + dd + [h0],
                   [Wg2, bg2r, wsr, Wdp, bdpr, Wdq, bdqr, P, PT])

    p = _agg_partials(hnu, src2d, dst2d, zero128)
    hnu1 = _tc_call(_t2_body, [(_NP, _H)],
                    [(p, 0), (p, 1)] + dd, [Wg1, bg1r])

    p = _agg_partials(hnu1, src2d, dst2d, zero128)
    out = _tc_call(_t5_body, [(_N, _OUT)],
                   [(p, 0), (p, 1)] + dd, [Wg2, bg2r, Wp1, bp1r, Wp2, bp2r])
    return out


# speedup vs baseline: 1.0033x; 1.0033x over previous
import functools

import jax
import jax.numpy as jnp
from jax import lax
from jax.experimental import pallas as pl
from jax.experimental.pallas import tpu as pltpu
from jax.experimental.pallas import tpu_sc as plsc

_N = 10000
_E = 320000
_H = 128
_OUT = 40
_POOL = 4

_NC = 2
_NS = 16
_NW = _NC * _NS
_CH = 128
_CPT = 80
_CPH = 40
_NCH = _NW * _CPT
_EP = _NCH * _CH
_NP = 10240
_ROWS_PER_TILE = _NP // _NS
_NPAD = _NP - _N

_BR = 1024
_G = _NP // _BR
_BC = _BR // _POOL


def _sc_mesh():
    return plsc.VectorSubcoreMesh(core_axis_name="c", subcore_axis_name="s")


def _deg_partials(dst2d, ones_rows, zero_rows):
    @functools.partial(
        pl.kernel,
        mesh=_sc_mesh(),
        out_type=jax.ShapeDtypeStruct((_NC, _NP, _H), jnp.float32),
        scratch_types=[
            pltpu.VMEM_SHARED((_NP, _H), jnp.float32),
            pltpu.VMEM((_CPT, _CH), jnp.int32),
            pltpu.VMEM((_CH, _H), jnp.float32),
            pltpu.SemaphoreType.DMA,
        ],
    )
    def k(dst_hbm, ones_hbm, z_hbm, out_hbm, acc, dst_v, ones_v, sem):
        c = lax.axis_index("c")
        s = lax.axis_index("s")
        w = s * _NC + c
        row0 = s * _ROWS_PER_TILE
        ch0 = w * _CPT
        pltpu.sync_copy(z_hbm, acc.at[pl.ds(row0, _ROWS_PER_TILE)])
        pltpu.sync_copy(dst_hbm.at[pl.ds(ch0, _CPT)], dst_v)
        pltpu.sync_copy(ones_hbm, ones_v)
        plsc.subcore_barrier()

        def body(j, carry):
            pltpu.async_copy(ones_v, acc.at[dst_v.at[j]], sem, add=True)

            @pl.when(j >= 4)
            def _():
                pltpu.make_async_copy(ones_v, acc.at[dst_v.at[j - 4]],
                                      sem).wait()

            return carry

        lax.fori_loop(0, _CPT, body, 0)
        for t in range(4):
            pltpu.make_async_copy(ones_v, acc.at[dst_v.at[_CPT - 4 + t]],
                                  sem).wait()
        plsc.subcore_barrier()
        pltpu.sync_copy(acc.at[pl.ds(row0, _ROWS_PER_TILE)],
                        out_hbm.at[c, pl.ds(row0, _ROWS_PER_TILE)])

    return k(dst2d, ones_rows, zero_rows)


def _agg_partials(hn, src2d, dst2d, zero_rows):
    @functools.partial(
        pl.kernel,
        mesh=_sc_mesh(),
        out_type=jax.ShapeDtypeStruct((_NC, _NP, _H), jnp.float32),
        scratch_types=[
            pltpu.VMEM_SHARED((_NP, _H), jnp.float32),
            pltpu.VMEM((_CPH, _CH), jnp.int32),
            pltpu.VMEM((_CPH, _CH), jnp.int32),
            pltpu.VMEM((_CH, _H), jnp.float32),
            pltpu.VMEM((_CH, _H), jnp.float32),
            pltpu.SemaphoreType.DMA,
            pltpu.SemaphoreType.DMA,
        ],
    )
    def k(hn_hbm, src_hbm, dst_hbm, z_hbm, out_hbm, acc, src_v, dst_v,
          rows0, rows1, sem0, sem1):
        c = lax.axis_index("c")
        s = lax.axis_index("s")
        w = s * _NC + c
        row0 = s * _ROWS_PER_TILE

        rows = (rows0, rows1)
        sems = (sem0, sem1)

        def gath(j, b):
            return pltpu.make_async_copy(hn_hbm.at[src_v.at[j]], rows[b],
                                         sems[b])

        for ph in range(2):
            ch0 = w * _CPT + ph * _CPH
            pltpu.sync_copy(src_hbm.at[pl.ds(ch0, _CPH)], src_v)
            pltpu.sync_copy(dst_hbm.at[pl.ds(ch0, _CPH)], dst_v)

            gath(0, 0).start()
            gath(1, 1).start()

            if ph == 0:
                pltpu.sync_copy(z_hbm, acc.at[pl.ds(row0, _ROWS_PER_TILE)])
                plsc.subcore_barrier()

            def body(i, carry):
                for b in range(2):
                    j = 2 * i + b
                    gath(j, b).wait()
                    pltpu.sync_copy(rows[b], acc.at[dst_v.at[j]], add=True)

                    @pl.when(j + 2 < _CPH)
                    def _():
                        gath(j + 2, b).start()

                return carry

            lax.fori_loop(0, _CPH // 2, body, 0)
        plsc.subcore_barrier()
        pltpu.sync_copy(acc.at[pl.ds(row0, _ROWS_PER_TILE)],
                        out_hbm.at[c, pl.ds(row0, _ROWS_PER_TILE)])

    return k(hn, src2d, dst2d, zero_rows)


def _row_spec(d):
    return pl.BlockSpec((_BR, d), lambda i: (i, 0))


def _plane_spec(d, c):
    return pl.BlockSpec((1, _BR, d), lambda i, _c=c: (_c, i, 0))


def _full_spec(shape):
    nd = len(shape)
    return pl.BlockSpec(shape, lambda i: (0,) * nd)


def _tc_call(body, out_shapes, row_args, full_args):
    in_specs = []
    arrs = []
    for a in row_args:
        if isinstance(a, tuple):
            arr, plane = a
            in_specs.append(_plane_spec(arr.shape[2], plane))
            arrs.append(arr)
        else:
            in_specs.append(_row_spec(a.shape[1]))
            arrs.append(a)
    row_args = arrs
    in_specs += [_full_spec(a.shape) for a in full_args]
    out_specs = [_row_spec(s[1]) for s in out_shapes]
    out_shape = [jax.ShapeDtypeStruct(s, jnp.float32) for s in out_shapes]
    single = len(out_shapes) == 1
    res = pl.pallas_call(
        body,
        grid=(_G,),
        in_specs=in_specs,
        out_specs=out_specs[0] if single else out_specs,
        out_shape=out_shape[0] if single else out_shape,
        compiler_params=pltpu.CompilerParams(
            dimension_semantics=("arbitrary",)),
    )(*row_args, *full_args)
    return res


def _nrm(d0, d1):
    return lax.rsqrt(d0[0, :, 0:1] + d1[0, :, 0:1] + 1.0)


def _psum(p0, p1):
    return p0[0] + p1[0]


def _mm(a, b):
    return jnp.dot(a, b, preferred_element_type=jnp.float32)


def _t1_body(x, d0, d1, W1, b1, W2, b2, h0_o, hn_o):
    nrm = _nrm(d0[...], d1[...])
    t = jnp.maximum(_mm(x[...], W1[...]) + b1[...], 0.0)
    h0 = _mm(t, W2[...]) + b2[...]
    h0_o[...] = h0
    hn_o[...] = h0 * nrm


def _t2_body(p0, p1, d0, d1, W, b, hn_o):
    nrm = _nrm(d0[...], d1[...])
    agg = _psum(p0[...], p1[...]) * nrm
    h = jnp.maximum(_mm(agg, W[...]) + b[...], 0.0)
    hn_o[...] = h * nrm


def _t3_body(p0, p1, d0, d1, h0, Wg2, bg2, ws, Wdp, bdp, Wdq, bdq, P, PT,
             hn_o):
    nrm = _nrm(d0[...], d1[...])
    agg = _psum(p0[...], p1[...]) * nrm
    hg = jnp.maximum(_mm(agg, Wg2[...]) + bg2[...], 0.0)
    sc = _mm(hg, ws[...])
    ns = 1.0 / (1.0 + jnp.exp(-sc))
    hd = jnp.maximum(_mm(hg, Wdp[...]) + bdp[...], 0.0)
    hp_pre = _mm(P[...], hd)
    ps = _mm(P[...], ns) * 0.25
    hp = jnp.maximum(_mm(hp_pre, Wdq[...]) + bdq[...], 0.0) * ps
    hu = _mm(PT[...], hp) + h0[...]
    hn_o[...] = hu * nrm


def _t5_body(p0, p1, d0, d1, W, b, Wp1, bp1, Wp2, bp2, out_o):
    nrm = _nrm(d0[...], d1[...])
    agg = _psum(p0[...], p1[...]) * nrm
    h = jnp.maximum(_mm(agg, W[...]) + b[...], 0.0)
    t = jnp.maximum(_mm(h, Wp1[...]) + bp1[...], 0.0)
    out_o[...] = _mm(t, Wp2[...]) + bp2[...]


def kernel(x, edge_index, W1, b1, W2, b2, Wg1, bg1, Wg2, bg2, w_score,
           Wdp, bdp, Wdq, bdq, Wp1, bp1, Wp2, bp2):
    src = edge_index[0].astype(jnp.int32)
    dst = edge_index[1].astype(jnp.int32)

    pad = _N + (jnp.arange(_EP - _E, dtype=jnp.int32) % _NPAD)
    src2d = jnp.concatenate([src, pad]).reshape(_NCH, _CH)
    dst2d = jnp.concatenate([dst, pad]).reshape(_NCH, _CH)

    ones_rows = jnp.ones((_CH, _H), jnp.float32)
    zero128 = jnp.zeros((_ROWS_PER_TILE, _H), jnp.float32)

    rows = jnp.arange(_BC)[:, None]
    cols = jnp.arange(_BR)[None, :]
    P = (cols // _POOL == rows).astype(jnp.float32)
    PT = P.T

    b1r = b1.reshape(1, _H)
    b2r = b2.reshape(1, _H)
    bg1r = bg1.reshape(1, _H)
    bg2r = bg2.reshape(1, _H)
    bdpr = bdp.reshape(1, _H)
    bdqr = bdq.reshape(1, _H)
    bp1r = bp1.reshape(1, _H)
    bp2r = bp2.reshape(1, _OUT)
    wsr = w_score.reshape(_H, 1)

    degp = _deg_partials(dst2d, ones_rows, zero128)
    degs = degp[:, :, :8]
    dd = [(degs, 0), (degs, 1)]

    h0, hn0 = _tc_call(_t1_body, [(_NP, _H), (_NP, _H)],
                       [x] + dd, [W1, b1r, W2, b2r])

    p = _agg_partials(hn0, src2d, dst2d, zero128)
    hn1 = _tc_call(_t2_body, [(_NP, _H)],
                   [(p, 0), (p, 1)] + dd, [Wg1, bg1r])

    p = _agg_partials(hn1, src2d, dst2d, zero128)
    hnu = _tc_call(_t3_body, [(_NP, _H)],
                   [(p, 0), (p, 1)] + dd + [h0],
                   [Wg2, bg2r, wsr, Wdp, bdpr, Wdq, bdqr, P, PT])

    p = _agg_partials(hnu, src2d, dst2d, zero128)
    hnu1 = _tc_call(_t2_body, [(_NP, _H)],
                    [(p, 0), (p, 1)] + dd, [Wg1, bg1r])

    p = _agg_partials(hnu1, src2d, dst2d, zero128)
    out = _tc_call(_t5_body, [(_N, _OUT)],
                   [(p, 0), (p, 1)] + dd, [Wg2, bg2r, Wp1, bp1r, Wp2, bp2r])
    return out
